# Initial kernel scaffold; baseline (speedup 1.0000x reference)
import jax
import jax.numpy as jnp
from jax.experimental import pallas as pl


def kernel(x, edge_index, edge_attr, batch, We0, W0, b0, We1, W1, b1, Wl0, bl0, Wl1, bl1, Wemb, bemb, Wout, bout):
    raise NotImplementedError("write your pallas kernel here")


# speedup vs baseline: 1.4304x; 1.4304x over previous
import functools

import jax
import jax.numpy as jnp
from jax import lax
from jax.experimental import pallas as pl
from jax.experimental.pallas import tpu as pltpu
from jax.experimental.pallas import tpu_sc as plsc

N = 10000
E = 160000
D = 256
DE = 16
H = 512
G = 64
T = 10

DH = D // 2
N_PAD = 10112
ROWS_PER_TILE = N_PAD // 16
CHUNK = 128
N_CHUNKS = E // CHUNK
N_TILES = 16


def _edge_proj_body(ea_ref, we0_ref, we1_ref, o00, o01, o10, o11):
    ea = ea_ref[...]
    p0 = jnp.dot(ea, we0_ref[...], preferred_element_type=jnp.float32)
    p1 = jnp.dot(ea, we1_ref[...], preferred_element_type=jnp.float32)
    o00[...] = p0[:, :DH]
    o01[...] = p0[:, DH:]
    o10[...] = p1[:, :DH]
    o11[...] = p1[:, DH:]


def _edge_proj(edge_attr, We0, We1):
    BE = 2000
    grid = (E // BE,)
    out = jax.ShapeDtypeStruct((E, DH), jnp.float32)
    return pl.pallas_call(
        _edge_proj_body,
        grid=grid,
        in_specs=[
            pl.BlockSpec((BE, DE), lambda i: (i, 0)),
            pl.BlockSpec((DE, D), lambda i: (0, 0)),
            pl.BlockSpec((DE, D), lambda i: (0, 0)),
        ],
        out_specs=[pl.BlockSpec((BE, DH), lambda i: (i, 0))] * 4,
        out_shape=[out, out, out, out],
    )(edge_attr, We0, We1)


def _sc_layer_body(x0, x1, ep0, ep1, src, dst, zeros, out0, out1,
                   sidx, didx, epb, xsb, agg, sem):
    c = lax.axis_index("c")
    s = lax.axis_index("s")

    pltpu.sync_copy(zeros, agg.at[pl.ds(s * ROWS_PER_TILE, ROWS_PER_TILE)])
    plsc.subcore_barrier()

    def do_edges(xh, eph):
        cnt = (N_CHUNKS - s + N_TILES - 1) // N_TILES

        def body(i, carry):
            off = (s + i * N_TILES) * CHUNK
            pltpu.sync_copy(src.at[pl.ds(off, CHUNK)], sidx.at[0])
            pltpu.sync_copy(dst.at[pl.ds(off, CHUNK)], didx.at[0])
            pltpu.sync_copy(eph.at[pl.ds(off, CHUNK)], epb)
            pltpu.async_copy(xh.at[sidx.at[0]], xsb, sem).wait()

            def crow(r, carry2):
                for k in range(DH // 16):
                    sl = pl.ds(k * 16, 16)
                    v = xsb[r, sl] + epb[r, sl]
                    xsb[r, sl] = jnp.maximum(v, 0.0)
                return carry2

            lax.fori_loop(0, CHUNK, crow, 0, unroll=2)
            pltpu.sync_copy(xsb, agg.at[didx.at[0]], add=True)
            return carry

        lax.fori_loop(0, cnt, body, 0)

    @pl.when(c == 0)
    def _():
        do_edges(x0, ep0)

    @pl.when(c == 1)
    def _():
        do_edges(x1, ep1)

    plsc.subcore_barrier()
    rows = pl.ds(s * ROWS_PER_TILE, ROWS_PER_TILE)

    @pl.when(c == 0)
    def _():
        pltpu.sync_copy(agg.at[rows], out0.at[rows])

    @pl.when(c == 1)
    def _():
        pltpu.sync_copy(agg.at[rows], out1.at[rows])


def _sc_layer(x0, x1, ep0, ep1, src, dst, zeros):
    mesh = plsc.VectorSubcoreMesh(core_axis_name="c", subcore_axis_name="s")
    out = jax.ShapeDtypeStruct((N_PAD, DH), jnp.float32)
    f = pl.kernel(
        _sc_layer_body,
        out_type=[out, out],
        mesh=mesh,
        scratch_types=[
            pltpu.VMEM((1, CHUNK), jnp.int32),
            pltpu.VMEM((1, CHUNK), jnp.int32),
            pltpu.VMEM((CHUNK, DH), jnp.float32),
            pltpu.VMEM((CHUNK, DH), jnp.float32),
            pltpu.VMEM_SHARED((N_PAD, DH), jnp.float32),
            pltpu.SemaphoreType.DMA,
        ],
    )
    return f(x0, x1, ep0, ep1, src, dst, zeros)


def _node_update_body(a0, a1, x0, x1, w_ref, b_ref, h0, h1):
    u0 = a0[...] + x0[...]
    u1 = a1[...] + x1[...]
    w = w_ref[...]
    acc = jnp.dot(u0, w[:DH, :], preferred_element_type=jnp.float32)
    acc = acc + jnp.dot(u1, w[DH:, :], preferred_element_type=jnp.float32)
    h = jnp.maximum(acc + b_ref[...], 0.0)
    h0[...] = h[:, :DH]
    h1[...] = h[:, DH:]


def _node_update(agg0, agg1, x0, x1, W, b):
    BN = 2000
    grid = (N // BN,)
    out = jax.ShapeDtypeStruct((N, DH), jnp.float32)
    half = pl.BlockSpec((BN, DH), lambda i: (i, 0))
    return pl.pallas_call(
        _node_update_body,
        grid=grid,
        in_specs=[half, half, half, half,
                  pl.BlockSpec((D, D), lambda i: (0, 0)),
                  pl.BlockSpec((1, D), lambda i: (0, 0))],
        out_specs=[half, half],
        out_shape=[out, out],
    )(agg0, agg1, x0, x1, W, b)


def _pool_mlp_body(batch_ref, h0, h1, wl0, bl0, wl1, bl1, wemb, bemb,
                   wout, bout, out_ref, acc):
    i = pl.program_id(0)

    @pl.when(i == 0)
    def _():
        acc[...] = jnp.zeros_like(acc)

    bi = batch_ref[0, 0, :]
    gid = lax.broadcasted_iota(jnp.int32, (G, bi.shape[0]), 0)
    mask = (gid == bi[None, :]).astype(jnp.float32)
    h = jnp.concatenate([h0[...], h1[...]], axis=1)
    acc[...] += jnp.dot(mask, h, preferred_element_type=jnp.float32)

    @pl.when(i == pl.num_programs(0) - 1)
    def _():
        p = acc[...]
        a = jnp.maximum(jnp.dot(p, wl0[...], preferred_element_type=jnp.float32) + bl0[...], 0.0)
        a = jnp.maximum(jnp.dot(a, wl1[...], preferred_element_type=jnp.float32) + bl1[...], 0.0)
        e = jnp.dot(a, wemb[...], preferred_element_type=jnp.float32) + bemb[...]
        out_ref[...] = jnp.dot(e, wout[...], preferred_element_type=jnp.float32) + bout[...]


def _pool_mlp(batch2d, h0, h1, Wl0, bl0, Wl1, bl1, Wemb, bemb, Wout_p, bout_p):
    BN = 2000
    grid = (N // BN,)
    half = pl.BlockSpec((BN, DH), lambda i: (i, 0))
    full = lambda r, c: pl.BlockSpec((r, c), lambda i: (0, 0))
    return pl.pallas_call(
        _pool_mlp_body,
        grid=grid,
        in_specs=[pl.BlockSpec((1, 1, BN), lambda i: (i, 0, 0)),
                  half, half,
                  full(D, H), full(1, H),
                  full(H, H), full(1, H),
                  full(H, H), full(1, H),
                  full(H, 128), full(1, 128)],
        out_specs=pl.BlockSpec((G, 128), lambda i: (0, 0)),
        out_shape=jax.ShapeDtypeStruct((G, 128), jnp.float32),
        scratch_shapes=[pltpu.VMEM((G, D), jnp.float32)],
    )(batch2d, h0, h1, Wl0, bl0, Wl1, bl1, Wemb, bemb, Wout_p, bout_p)


def kernel(x, edge_index, edge_attr, batch, We0, W0, b0, We1, W1, b1,
           Wl0, bl0, Wl1, bl1, Wemb, bemb, Wout, bout):
    src = edge_index[0]
    dst = edge_index[1]
    x0 = x[:, :DH]
    x1 = x[:, DH:]
    zeros = jnp.zeros((ROWS_PER_TILE, DH), jnp.float32)

    ep00, ep01, ep10, ep11 = _edge_proj(edge_attr, We0, We1)

    a0, a1 = _sc_layer(x0, x1, ep00, ep01, src, dst, zeros)
    h0, h1 = _node_update(a0[:N], a1[:N], x0, x1, W0, b0.reshape(1, D))

    a0, a1 = _sc_layer(h0, h1, ep10, ep11, src, dst, zeros)
    h0, h1 = _node_update(a0[:N], a1[:N], h0, h1, W1, b1.reshape(1, D))

    Wout_p = jnp.pad(Wout, ((0, 0), (0, 128 - T)))
    bout_p = jnp.pad(bout, (0, 128 - T)).reshape(1, 128)
    out = _pool_mlp(batch.reshape(N // 2000, 1, 2000), h0, h1,
                    Wl0, bl0.reshape(1, H), Wl1, bl1.reshape(1, H),
                    Wemb, bemb.reshape(1, H), Wout_p, bout_p)
    return out[:, :T]
